# baseline (speedup 1.0000x reference)
import functools

import jax
import jax.numpy as jnp
from jax import lax
from jax.experimental import pallas as pl
from jax.experimental.pallas import tpu as pltpu
from jax.experimental.pallas import tpu_sc as plsc

_NUM_CLASSES = 1000
_DATASET_LEN = 100000
_BATCH = 4096
_TOTAL_EPOCHS = 300
_ALPHA_T = 0.8
_EPOCH_CONST = 5
_ALPHA = _ALPHA_T * ((_EPOCH_CONST + 1) / _TOTAL_EPOCHS)

_NC = 2
_NS = 16
_NW = _NC * _NS
_BLK = 512
_NBLK = _BATCH // _BLK
_LANES = 16
_NVEC = _BATCH // _LANES


def _prep_body(idx_col_ref, idx_row_ref, outT_ref, npT_ref, win_ref):
    x = outT_ref[...]
    m = jnp.max(x, axis=0, keepdims=True)
    e = jnp.exp(x - m)
    npT_ref[...] = e / jnp.sum(e, axis=0, keepdims=True)

    mine = idx_col_ref[...]
    alls = idx_row_ref[...]
    eq = mine == alls
    slot = lax.broadcasted_iota(jnp.int32, (_BLK, _BATCH), 1)
    win = jnp.max(jnp.where(eq, slot, -1), axis=1, keepdims=True)
    myslot = (lax.broadcasted_iota(jnp.int32, (_BLK, 1), 0)
              + pl.program_id(0) * _BLK)
    win_ref[...] = jnp.where(win == myslot, mine, -1 - mine)


def _tc_prep(idx_col, idx_row, outputsT):
    return pl.pallas_call(
        _prep_body,
        grid=(_NBLK,),
        in_specs=[
            pl.BlockSpec((_BLK, 1), lambda i: (i, 0)),
            pl.BlockSpec((1, _BATCH), lambda i: (0, 0)),
            pl.BlockSpec((_NUM_CLASSES, _BLK), lambda i: (0, i)),
        ],
        out_specs=[
            pl.BlockSpec((_NUM_CLASSES, _BLK), lambda i: (0, i)),
            pl.BlockSpec((_BLK, 1), lambda i: (i, 0)),
        ],
        out_shape=[
            jax.ShapeDtypeStruct((_NUM_CLASSES, _BATCH), jnp.float32),
            jax.ShapeDtypeStruct((_BATCH, 1), jnp.int32),
        ],
    )(idx_col, idx_row, outputsT)


def _loss_body(alpha_ref, outT_ref, tgtT_ref, gatT_ref, loss_ref):
    a = alpha_ref[0, 0]
    x = outT_ref[...]
    m = jnp.max(x, axis=0, keepdims=True)
    e = jnp.exp(x - m)
    logp = (x - m) - jnp.log(jnp.sum(e, axis=0, keepdims=True))
    soft = (1.0 - a) * tgtT_ref[...] + a * gatT_ref[...]
    part = -jnp.sum(soft * logp) * (1.0 / _BATCH)

    @pl.when(pl.program_id(0) == 0)
    def _():
        loss_ref[0, 0] = 0.0

    loss_ref[0, 0] += part


def _tc_loss(alpha, outputsT, targetsT, gatheredT):
    return pl.pallas_call(
        _loss_body,
        grid=(_NBLK,),
        in_specs=[
            pl.BlockSpec(memory_space=pltpu.SMEM),
            pl.BlockSpec((_NUM_CLASSES, _BLK), lambda i: (0, i)),
            pl.BlockSpec((_NUM_CLASSES, _BLK), lambda i: (0, i)),
            pl.BlockSpec((_NUM_CLASSES, _BLK), lambda i: (0, i)),
        ],
        out_specs=pl.BlockSpec(memory_space=pltpu.SMEM),
        out_shape=jax.ShapeDtypeStruct((1, 1), jnp.float32),
    )(alpha, outputsT, targetsT, gatheredT)


_MESH = plsc.VectorSubcoreMesh(core_axis_name="c", subcore_axis_name="s")

_CHUNK_LO = (0, 33280, 66560)
_CHUNK_SZ = (33280, 33280, 33440)


@functools.partial(
    pl.kernel,
    mesh=_MESH,
    out_type=(
        jax.ShapeDtypeStruct((_NUM_CLASSES, _DATASET_LEN), jnp.float32),
        jax.ShapeDtypeStruct((_NUM_CLASSES, _BATCH), jnp.float32),
    ),
    compiler_params=pltpu.CompilerParams(needs_layout_passes=False),
    scratch_types=[
        pltpu.VMEM((_CHUNK_SZ[0],), jnp.float32),
        pltpu.VMEM((_CHUNK_SZ[1],), jnp.float32),
        pltpu.VMEM((_CHUNK_SZ[2],), jnp.float32),
        pltpu.VMEM((_BATCH,), jnp.float32),
        pltpu.VMEM((_BATCH,), jnp.float32),
        pltpu.VMEM((_BATCH,), jnp.int32),
        pltpu.VMEM((_BATCH,), jnp.int32),
        pltpu.VMEM((_BATCH,), jnp.int32),
        pltpu.VMEM((_BATCH,), jnp.int32),
        pltpu.SemaphoreType.DMA,
        pltpu.SemaphoreType.DMA,
        pltpu.SemaphoreType.DMA,
        pltpu.SemaphoreType.DMA,
        pltpu.SemaphoreType.DMA,
        pltpu.SemaphoreType.DMA,
        pltpu.SemaphoreType.DMA,
        pltpu.SemaphoreType.DMA,
    ],
)
def _sc_fused(apT_hbm, npT_hbm, enc_hbm, outT_hbm, gatT_hbm,
              b0, b1, b2, gar_v, npr_v, enc_v, ge0_v, ge1_v, ge2_v,
              si0, si1, si2, so0, so1, so2, snp, sgr):
    bufs = (b0, b1, b2)
    gencs = (ge0_v, ge1_v, ge2_v)
    sin = (si0, si1, si2)
    sout = (so0, so1, so2)
    w = lax.axis_index("s") * _NC + lax.axis_index("c")
    pltpu.sync_copy(enc_hbm, enc_v)
    nrows = jnp.where(w < _NUM_CLASSES % _NW, _NUM_CLASSES // _NW + 1,
                      _NUM_CLASSES // _NW)

    for q in range(3):
        pltpu.async_copy(apT_hbm.at[w, pl.ds(_CHUNK_LO[q], _CHUNK_SZ[q])],
                         bufs[q], sin[q])
    pltpu.async_copy(npT_hbm.at[w], npr_v, snp)

    for q in range(3):
        lo = _CHUNK_LO[q]
        sz = _CHUNK_SZ[q]
        geq = gencs[q]

        def pre_body(j, _, lo=lo, sz=sz, geq=geq):
            sl = pl.ds(j * _LANES, _LANES)
            ev = enc_v[sl]
            winb = ev >= 0
            riv = jnp.where(winb, ev, -1 - ev)
            m = (riv >= lo) & (riv < lo + sz)
            lidx = riv - lo
            lidx = jnp.minimum(jnp.maximum(lidx, 0), sz - 1)
            packed = (lidx
                      + jnp.where(m, 65536, 0)
                      + jnp.where(m & winb, 131072, 0))
            geq[sl] = packed
            return 0

        lax.fori_loop(0, _NVEC, pre_body, 0)

    def row_body(t, _):
        c = w + _NW * t
        pltpu.make_async_copy(npT_hbm.at[c], npr_v, snp).wait()
        @pl.when(t > 0)
        def _():
            pltpu.make_async_copy(gar_v, gatT_hbm.at[c], sgr).wait()
        for q in range(3):
            lo = _CHUNK_LO[q]
            sz = _CHUNK_SZ[q]
            bq = bufs[q]
            pltpu.make_async_copy(
                apT_hbm.at[c, pl.ds(lo, sz)], bq, sin[q]).wait()

            geq = gencs[q]

            @plsc.parallel_loop(0, _NVEC)
            def _(j, bq=bq, q=q, geq=geq):
                sl = pl.ds(j * _LANES, _LANES)
                ge = geq[sl]
                lidx = ge & 65535
                m = (ge & 65536) != 0
                vals = plsc.load_gather(bq, [lidx], mask=m)
                if q == 0:
                    gar_v[sl] = vals
                else:
                    gar_v[sl] = jnp.where(m, vals, gar_v[sl])

            @plsc.parallel_loop(0, _NVEC)
            def _(j, bq=bq, geq=geq):
                sl = pl.ds(j * _LANES, _LANES)
                ge = geq[sl]
                lidx = ge & 65535
                sm = (ge & 131072) != 0
                sval = npr_v[sl]
                plsc.store_scatter(bq, [lidx], sval, mask=sm)
            pltpu.async_copy(bq, outT_hbm.at[c, pl.ds(lo, sz)], sout[q])
        pltpu.async_copy(gar_v, gatT_hbm.at[c], sgr)

        @pl.when(t + 1 < nrows)
        def _():
            cn = c + _NW
            pltpu.async_copy(npT_hbm.at[cn], npr_v, snp)
            for q in range(3):
                lo = _CHUNK_LO[q]
                sz = _CHUNK_SZ[q]
                pltpu.make_async_copy(
                    bufs[q], outT_hbm.at[c, pl.ds(lo, sz)],
                    sout[q]).wait()
                pltpu.async_copy(apT_hbm.at[cn, pl.ds(lo, sz)],
                                 bufs[q], sin[q])
        return 0

    lax.fori_loop(0, nrows, row_body, 0)

    clast = w + _NW * (nrows - 1)
    pltpu.make_async_copy(gar_v, gatT_hbm.at[clast], sgr).wait()
    for q in range(3):
        pltpu.make_async_copy(
            bufs[q], outT_hbm.at[clast, pl.ds(_CHUNK_LO[q], _CHUNK_SZ[q])],
            sout[q]).wait()


def kernel(samples, outputs, targets, all_predictions, input_indices, epoch):
    del samples
    alpha = jnp.where(jnp.asarray(epoch) == 0, 0.0, _ALPHA)
    alpha = jnp.asarray(alpha, jnp.float32).reshape(1, 1)

    apT = all_predictions.T
    outputsT = outputs.T
    targetsT = targets.T

    idx_col = input_indices.reshape(_BATCH, 1)
    idx_row = input_indices.reshape(1, _BATCH)

    newpT, enc = _tc_prep(idx_col, idx_row, outputsT)
    outT, gatheredT = _sc_fused(apT, newpT, enc.reshape(_BATCH))
    loss = _tc_loss(alpha, outputsT, targetsT, gatheredT)[0, 0]
    return loss, outT.T


# speedup vs baseline: 1.0855x; 1.0855x over previous
import functools

import jax
import jax.numpy as jnp
from jax import lax
from jax.experimental import pallas as pl
from jax.experimental.pallas import tpu as pltpu
from jax.experimental.pallas import tpu_sc as plsc

_NUM_CLASSES = 1000
_DATASET_LEN = 100000
_BATCH = 4096
_TOTAL_EPOCHS = 300
_ALPHA_T = 0.8
_EPOCH_CONST = 5
_ALPHA = _ALPHA_T * ((_EPOCH_CONST + 1) / _TOTAL_EPOCHS)

_NC = 2
_NS = 16
_NW = _NC * _NS
_BLK = 512
_NBLK = _BATCH // _BLK
_LANES = 16
_NVEC = _BATCH // _LANES


def _prep_body(idx_col_ref, idx_row_ref, outT_ref, npT_ref, win_ref):
    x = outT_ref[...]
    m = jnp.max(x, axis=0, keepdims=True)
    e = jnp.exp(x - m)
    npT_ref[...] = e / jnp.sum(e, axis=0, keepdims=True)

    mine = idx_col_ref[...]
    alls = idx_row_ref[...]
    eq = mine == alls
    slot = lax.broadcasted_iota(jnp.int32, (_BLK, _BATCH), 1)
    win = jnp.max(jnp.where(eq, slot, -1), axis=1, keepdims=True)
    myslot = (lax.broadcasted_iota(jnp.int32, (_BLK, 1), 0)
              + pl.program_id(0) * _BLK)
    win_ref[...] = jnp.where(win == myslot, mine, -1 - mine)


def _tc_prep(idx_col, idx_row, outputsT):
    return pl.pallas_call(
        _prep_body,
        grid=(_NBLK,),
        in_specs=[
            pl.BlockSpec((_BLK, 1), lambda i: (i, 0)),
            pl.BlockSpec((1, _BATCH), lambda i: (0, 0)),
            pl.BlockSpec((_NUM_CLASSES, _BLK), lambda i: (0, i)),
        ],
        out_specs=[
            pl.BlockSpec((_NUM_CLASSES, _BLK), lambda i: (0, i)),
            pl.BlockSpec((_BLK, 1), lambda i: (i, 0)),
        ],
        out_shape=[
            jax.ShapeDtypeStruct((_NUM_CLASSES, _BATCH), jnp.float32),
            jax.ShapeDtypeStruct((_BATCH, 1), jnp.int32),
        ],
    )(idx_col, idx_row, outputsT)


def _loss_body(alpha_ref, outT_ref, tgtT_ref, gatT_ref, loss_ref):
    a = alpha_ref[0, 0]
    x = outT_ref[...]
    m = jnp.max(x, axis=0, keepdims=True)
    e = jnp.exp(x - m)
    logp = (x - m) - jnp.log(jnp.sum(e, axis=0, keepdims=True))
    soft = (1.0 - a) * tgtT_ref[...] + a * gatT_ref[...]
    part = -jnp.sum(soft * logp) * (1.0 / _BATCH)

    @pl.when(pl.program_id(0) == 0)
    def _():
        loss_ref[0, 0] = 0.0

    loss_ref[0, 0] += part


def _tc_loss(alpha, outputsT, targetsT, gatheredT):
    return pl.pallas_call(
        _loss_body,
        grid=(_NBLK,),
        in_specs=[
            pl.BlockSpec(memory_space=pltpu.SMEM),
            pl.BlockSpec((_NUM_CLASSES, _BLK), lambda i: (0, i)),
            pl.BlockSpec((_NUM_CLASSES, _BLK), lambda i: (0, i)),
            pl.BlockSpec((_NUM_CLASSES, _BLK), lambda i: (0, i)),
        ],
        out_specs=pl.BlockSpec(memory_space=pltpu.SMEM),
        out_shape=jax.ShapeDtypeStruct((1, 1), jnp.float32),
    )(alpha, outputsT, targetsT, gatheredT)


_MESH = plsc.VectorSubcoreMesh(core_axis_name="c", subcore_axis_name="s")

_CHUNK_LO = (0, 49920)
_CHUNK_SZ = (49920, 50080)


@functools.partial(
    pl.kernel,
    mesh=_MESH,
    out_type=(
        jax.ShapeDtypeStruct((_NUM_CLASSES, _DATASET_LEN), jnp.float32),
        jax.ShapeDtypeStruct((_NUM_CLASSES, _BATCH), jnp.float32),
    ),
    compiler_params=pltpu.CompilerParams(needs_layout_passes=False),
    scratch_types=[
        pltpu.VMEM((_CHUNK_SZ[0],), jnp.float32),
        pltpu.VMEM((_CHUNK_SZ[1],), jnp.float32),
        pltpu.VMEM((_BATCH,), jnp.float32),
        pltpu.VMEM((_BATCH,), jnp.float32),
        pltpu.VMEM((_BATCH,), jnp.int32),
        pltpu.VMEM((_BATCH,), jnp.int32),
        pltpu.VMEM((_BATCH,), jnp.int32),
        pltpu.SemaphoreType.DMA,
        pltpu.SemaphoreType.DMA,
        pltpu.SemaphoreType.DMA,
        pltpu.SemaphoreType.DMA,
        pltpu.SemaphoreType.DMA,
        pltpu.SemaphoreType.DMA,
    ],
)
def _sc_fused(apT_hbm, npT_hbm, enc_hbm, outT_hbm, gatT_hbm,
              b0, b1, gar_v, npr_v, enc_v, ge0_v, ge1_v, si0, si1, so0, so1,
              snp, sgr):
    bufs = (b0, b1)
    gencs = (ge0_v, ge1_v)
    sin = (si0, si1)
    sout = (so0, so1)
    w = lax.axis_index("s") * _NC + lax.axis_index("c")
    pltpu.sync_copy(enc_hbm, enc_v)
    nrows = jnp.where(w < _NUM_CLASSES % _NW, _NUM_CLASSES // _NW + 1,
                      _NUM_CLASSES // _NW)

    for q in range(2):
        pltpu.async_copy(apT_hbm.at[w, pl.ds(_CHUNK_LO[q], _CHUNK_SZ[q])],
                         bufs[q], sin[q])
    pltpu.async_copy(npT_hbm.at[w], npr_v, snp)

    for q in range(2):
        lo = _CHUNK_LO[q]
        sz = _CHUNK_SZ[q]
        geq = gencs[q]

        def pre_body(j, _, lo=lo, sz=sz, geq=geq):
            sl = pl.ds(j * _LANES, _LANES)
            ev = enc_v[sl]
            winb = ev >= 0
            riv = jnp.where(winb, ev, -1 - ev)
            m = (riv >= lo) & (riv < lo + sz)
            lidx = riv - lo
            lidx = jnp.minimum(jnp.maximum(lidx, 0), sz - 1)
            packed = (lidx
                      + jnp.where(m, 65536, 0)
                      + jnp.where(m & winb, 131072, 0))
            geq[sl] = packed
            return 0

        lax.fori_loop(0, _NVEC, pre_body, 0)

    def row_body(t, _):
        c = w + _NW * t
        pltpu.make_async_copy(npT_hbm.at[c], npr_v, snp).wait()
        @pl.when(t > 0)
        def _():
            pltpu.make_async_copy(gar_v, gatT_hbm.at[c], sgr).wait()
        for q in range(2):
            lo = _CHUNK_LO[q]
            sz = _CHUNK_SZ[q]
            bq = bufs[q]
            pltpu.make_async_copy(
                apT_hbm.at[c, pl.ds(lo, sz)], bq, sin[q]).wait()

            geq = gencs[q]

            @plsc.parallel_loop(0, _NVEC)
            def _(j, bq=bq, q=q, geq=geq):
                sl = pl.ds(j * _LANES, _LANES)
                ge = geq[sl]
                lidx = ge & 65535
                m = (ge & 65536) != 0
                vals = plsc.load_gather(bq, [lidx], mask=m)
                if q == 0:
                    gar_v[sl] = vals
                else:
                    gar_v[sl] = jnp.where(m, vals, gar_v[sl])

            @plsc.parallel_loop(0, _NVEC)
            def _(j, bq=bq, geq=geq):
                sl = pl.ds(j * _LANES, _LANES)
                ge = geq[sl]
                lidx = ge & 65535
                sm = (ge & 131072) != 0
                sval = npr_v[sl]
                plsc.store_scatter(bq, [lidx], sval, mask=sm)
            pltpu.async_copy(bq, outT_hbm.at[c, pl.ds(lo, sz)], sout[q])
        pltpu.async_copy(gar_v, gatT_hbm.at[c], sgr)

        @pl.when(t + 1 < nrows)
        def _():
            cn = c + _NW
            pltpu.async_copy(npT_hbm.at[cn], npr_v, snp)
            for q in range(2):
                lo = _CHUNK_LO[q]
                sz = _CHUNK_SZ[q]
                pltpu.make_async_copy(
                    bufs[q], outT_hbm.at[c, pl.ds(lo, sz)],
                    sout[q]).wait()
                pltpu.async_copy(apT_hbm.at[cn, pl.ds(lo, sz)],
                                 bufs[q], sin[q])
        return 0

    lax.fori_loop(0, nrows, row_body, 0)

    clast = w + _NW * (nrows - 1)
    pltpu.make_async_copy(gar_v, gatT_hbm.at[clast], sgr).wait()
    for q in range(2):
        pltpu.make_async_copy(
            bufs[q], outT_hbm.at[clast, pl.ds(_CHUNK_LO[q], _CHUNK_SZ[q])],
            sout[q]).wait()


def kernel(samples, outputs, targets, all_predictions, input_indices, epoch):
    del samples
    alpha = jnp.where(jnp.asarray(epoch) == 0, 0.0, _ALPHA)
    alpha = jnp.asarray(alpha, jnp.float32).reshape(1, 1)

    apT = all_predictions.T
    outputsT = outputs.T
    targetsT = targets.T

    idx_col = input_indices.reshape(_BATCH, 1)
    idx_row = input_indices.reshape(1, _BATCH)

    newpT, enc = _tc_prep(idx_col, idx_row, outputsT)
    outT, gatheredT = _sc_fused(apT, newpT, enc.reshape(_BATCH))
    loss = _tc_loss(alpha, outputsT, targetsT, gatheredT)[0, 0]
    return loss, outT.T
